# baseline (speedup 1.0000x reference)
import functools

import jax
import jax.numpy as jnp
from jax import lax
from jax.experimental import pallas as pl
from jax.experimental.pallas import tpu as pltpu

G = 8
BT = 128
BN = 1024


def _tile_group(cumt, t):
    g = 0
    for d in range(G):
        g = g + jnp.where(cumt[d] <= t, 1, 0)
    return jnp.minimum(g, G - 1)


def _grouped_body(cumt, x_ref, qw0, qw1, s0, s1, bw, bias_ref, out_ref,
                  xbf, wbf):
    c = pl.program_id(0)
    t = pl.program_id(1)
    nc_s = pl.num_programs(0) // 2
    g = _tile_group(cumt, t)
    prev_g = _tile_group(cumt, jnp.maximum(t - 1, 0))
    new_g = (t == 0) | (g != prev_g)

    @pl.when((c == 0) & (t == 0))
    def _():
        xbf[...] = x_ref[...].astype(jnp.bfloat16)

    @pl.when(new_g & (c < nc_s))
    def _():
        scale = s0[0, 0, 0, :]
        wbf[...] = (bw[...] + scale[:, None] *
                    (qw0[0] - 8).astype(jnp.float32)).astype(jnp.bfloat16)

    @pl.when(new_g & (c >= nc_s))
    def _():
        scale = s1[0, 0, 0, :]
        wbf[...] = (bw[...] + scale[:, None] *
                    (qw1[0] - 8).astype(jnp.float32)).astype(jnp.bfloat16)

    xb = xbf[pl.ds(t * BT, BT), :]
    out_ref[...] = lax.dot_general(xb, wbf[...], (((1,), (1,)), ((), ())),
                                   preferred_element_type=jnp.float32
                                   ) + bias_ref[0]


@jax.jit
def kernel(x, base_W, bias, qweight0, qweight1, scales0, scales1, indices):
    T, D = x.shape
    NOUT = base_W.shape[0]
    SL = NOUT // 2
    nc = NOUT // BN
    nc_s = SL // BN
    TP = T + G * BT
    ntp = TP // BT

    onehot = (indices[:, None] == jnp.arange(G)[None, :]).astype(jnp.int32)
    sizes = jnp.sum(onehot, axis=0)
    rank = (jnp.cumsum(onehot, axis=0) - onehot)[jnp.arange(T), indices]
    tiles_g = (sizes + BT - 1) // BT
    cumt = jnp.cumsum(tiles_g).astype(jnp.int32)
    pad_off = (jnp.concatenate([jnp.zeros(1, jnp.int32), cumt[:-1]]) * BT)
    pos = pad_off[indices] + rank
    x_p = jnp.zeros((TP, D), x.dtype).at[pos].set(x, unique_indices=True)

    s0r = scales0.reshape(G, nc_s, 1, BN)
    s1r = scales1.reshape(G, nc_s, 1, BN)
    bias_r = bias.reshape(nc, 1, BN)

    grid_spec = pltpu.PrefetchScalarGridSpec(
        num_scalar_prefetch=1,
        grid=(nc, ntp),
        in_specs=[
            pl.BlockSpec((TP, D), lambda c, t, cumt: (0, 0)),
            pl.BlockSpec((1, BN, D),
                         lambda c, t, cumt: (
                             jnp.where(c < nc_s, _tile_group(cumt, t), 0),
                             jnp.where(c < nc_s, c, 0), 0)),
            pl.BlockSpec((1, BN, D),
                         lambda c, t, cumt: (
                             jnp.where(c >= nc_s, _tile_group(cumt, t), 0),
                             jnp.where(c >= nc_s, c - nc_s, 0), 0)),
            pl.BlockSpec((1, 1, 1, BN),
                         lambda c, t, cumt: (
                             jnp.where(c < nc_s, _tile_group(cumt, t), 0),
                             jnp.where(c < nc_s, c, 0), 0, 0)),
            pl.BlockSpec((1, 1, 1, BN),
                         lambda c, t, cumt: (
                             jnp.where(c >= nc_s, _tile_group(cumt, t), 0),
                             jnp.where(c >= nc_s, c - nc_s, 0), 0, 0)),
            pl.BlockSpec((BN, D), lambda c, t, cumt: (c, 0)),
            pl.BlockSpec((1, 1, BN), lambda c, t, cumt: (c, 0, 0)),
        ],
        out_specs=pl.BlockSpec((BT, BN), lambda c, t, cumt: (t, c)),
        scratch_shapes=[
            pltpu.VMEM((TP, D), jnp.bfloat16),
            pltpu.VMEM((BN, D), jnp.bfloat16),
        ],
    )

    out_p = pl.pallas_call(
        _grouped_body,
        grid_spec=grid_spec,
        out_shape=jax.ShapeDtypeStruct((TP, NOUT), jnp.float32),
        compiler_params=pltpu.CompilerParams(
            dimension_semantics=("arbitrary", "arbitrary")),
    )(cumt, x_p, qweight0, qweight1, s0r, s1r, base_W, bias_r)

    return jnp.take(out_p, pos, axis=0)
